# baseline (speedup 1.0000x reference)
import jax
import jax.numpy as jnp
from jax import lax
from jax.experimental import pallas as pl

DIM = 2048
NUM_EXPERTS = 16
TOP_K = 2
T = 16384
TT = 512


def _router_body(x_ref, w_ref, b_ref, ts_ref, se_ref, cnt_ref):
    i = pl.program_id(0)
    logits = lax.dot_general(
        w_ref[...], x_ref[...],
        dimension_numbers=(((1,), (1,)), ((), ())),
        preferred_element_type=jnp.float32,
    )
    scores = jax.nn.sigmoid(logits)
    biased = scores + b_ref[...]

    iota = lax.broadcasted_iota(jnp.int32, (NUM_EXPERTS, TT), 0)
    neg_inf = jnp.float32(-jnp.inf)

    m1 = jnp.max(biased, axis=0, keepdims=True)
    idx1 = jnp.min(jnp.where(biased == m1, iota, NUM_EXPERTS), axis=0, keepdims=True)
    sel1 = iota == idx1
    s1 = jnp.max(jnp.where(sel1, scores, neg_inf), axis=0, keepdims=True)

    biased2 = jnp.where(sel1, neg_inf, biased)
    m2 = jnp.max(biased2, axis=0, keepdims=True)
    idx2 = jnp.min(jnp.where(biased2 == m2, iota, NUM_EXPERTS), axis=0, keepdims=True)
    sel2 = iota == idx2
    s2 = jnp.max(jnp.where(sel2, scores, neg_inf), axis=0, keepdims=True)

    ts_ref[...] = jnp.concatenate([s1, s2], axis=0)
    se_ref[...] = jnp.concatenate([idx1, idx2], axis=0)

    onehot = sel1.astype(jnp.float32) + sel2.astype(jnp.float32)
    ones = jnp.ones((TT, 1), dtype=jnp.float32)
    cnt = lax.dot_general(
        onehot, ones,
        dimension_numbers=(((1,), (0,)), ((), ())),
        preferred_element_type=jnp.float32,
    )

    @pl.when(i == 0)
    def _init():
        cnt_ref[...] = cnt

    @pl.when(i > 0)
    def _acc():
        cnt_ref[...] += cnt


def kernel(x, W, expert_bias):
    bias2d = expert_bias.reshape(NUM_EXPERTS, 1)
    grid = (T // TT,)
    ts_t, se_t, counts = pl.pallas_call(
        _router_body,
        grid=grid,
        in_specs=[
            pl.BlockSpec((TT, DIM), lambda i: (i, 0)),
            pl.BlockSpec((NUM_EXPERTS, DIM), lambda i: (0, 0)),
            pl.BlockSpec((NUM_EXPERTS, 1), lambda i: (0, 0)),
        ],
        out_specs=[
            pl.BlockSpec((TOP_K, TT), lambda i: (0, i)),
            pl.BlockSpec((TOP_K, TT), lambda i: (0, i)),
            pl.BlockSpec((NUM_EXPERTS, 1), lambda i: (0, 0)),
        ],
        out_shape=[
            jax.ShapeDtypeStruct((TOP_K, T), jnp.float32),
            jax.ShapeDtypeStruct((TOP_K, T), jnp.int32),
            jax.ShapeDtypeStruct((NUM_EXPERTS, 1), jnp.float32),
        ],
    )(x, W, bias2d)
    return ts_t.T, se_t.T, counts.reshape(NUM_EXPERTS)


# speedup vs baseline: 1.0169x; 1.0169x over previous
import jax
import jax.numpy as jnp
from jax import lax
from jax.experimental import pallas as pl

DIM = 2048
NUM_EXPERTS = 16
TOP_K = 2
T = 16384
NSTREAM = 4
TT = 512
CHUNK = T // NSTREAM


def _top2(scores, biased):
    iota = lax.broadcasted_iota(jnp.int32, (NUM_EXPERTS, TT), 0)
    neg_inf = jnp.float32(-jnp.inf)
    m1 = jnp.max(biased, axis=0, keepdims=True)
    idx1 = jnp.min(jnp.where(biased == m1, iota, NUM_EXPERTS), axis=0, keepdims=True)
    sel1 = iota == idx1
    s1 = jnp.max(jnp.where(sel1, scores, neg_inf), axis=0, keepdims=True)
    biased2 = jnp.where(sel1, neg_inf, biased)
    m2 = jnp.max(biased2, axis=0, keepdims=True)
    idx2 = jnp.min(jnp.where(biased2 == m2, iota, NUM_EXPERTS), axis=0, keepdims=True)
    sel2 = iota == idx2
    s2 = jnp.max(jnp.where(sel2, scores, neg_inf), axis=0, keepdims=True)
    ts = jnp.concatenate([s1, s2], axis=0)
    se = jnp.concatenate([idx1, idx2], axis=0)
    onehot = sel1.astype(jnp.float32) + sel2.astype(jnp.float32)
    return ts, se, onehot


def _router_body(*refs):
    x_refs = refs[:NSTREAM]
    w_ref, b_ref = refs[NSTREAM], refs[NSTREAM + 1]
    ts_refs = refs[NSTREAM + 2:2 * NSTREAM + 2]
    se_refs = refs[2 * NSTREAM + 2:3 * NSTREAM + 2]
    cnt_ref = refs[3 * NSTREAM + 2]
    i = pl.program_id(0)

    w = w_ref[...]
    b = b_ref[...]
    ones = jnp.ones((TT, 1), dtype=jnp.float32)
    cnt = jnp.zeros((NUM_EXPERTS, 1), dtype=jnp.float32)
    for j in range(NSTREAM):
        logits = lax.dot_general(
            w, x_refs[j][...],
            dimension_numbers=(((1,), (1,)), ((), ())),
            preferred_element_type=jnp.float32,
        )
        scores = jax.nn.sigmoid(logits)
        ts, se, onehot = _top2(scores, scores + b)
        ts_refs[j][...] = ts
        se_refs[j][...] = se
        cnt = cnt + lax.dot_general(
            onehot, ones,
            dimension_numbers=(((1,), (0,)), ((), ())),
            preferred_element_type=jnp.float32,
        )

    @pl.when(i == 0)
    def _init():
        cnt_ref[...] = cnt

    @pl.when(i > 0)
    def _acc():
        cnt_ref[...] += cnt


def kernel(x, W, expert_bias):
    bias2d = expert_bias.reshape(NUM_EXPERTS, 1)
    nb = CHUNK // TT
    grid = (nb,)

    def x_spec(j):
        return pl.BlockSpec((TT, DIM), lambda i, j=j: (j * nb + i, 0))

    def o_spec(j):
        return pl.BlockSpec((TOP_K, TT), lambda i, j=j: (0, j * nb + i))

    outs = pl.pallas_call(
        _router_body,
        grid=grid,
        in_specs=[x_spec(j) for j in range(NSTREAM)]
        + [
            pl.BlockSpec((NUM_EXPERTS, DIM), lambda i: (0, 0)),
            pl.BlockSpec((NUM_EXPERTS, 1), lambda i: (0, 0)),
        ],
        out_specs=[o_spec(j) for j in range(NSTREAM)]
        + [o_spec(j) for j in range(NSTREAM)]
        + [pl.BlockSpec((NUM_EXPERTS, 1), lambda i: (0, 0))],
        out_shape=[jax.ShapeDtypeStruct((TOP_K, T), jnp.float32)] * NSTREAM
        + [jax.ShapeDtypeStruct((TOP_K, T), jnp.int32)] * NSTREAM
        + [jax.ShapeDtypeStruct((NUM_EXPERTS, 1), jnp.float32)],
    )(*([x] * NSTREAM), W, bias2d)

    ts_parts = outs[:NSTREAM]
    se_parts = outs[NSTREAM:2 * NSTREAM]
    counts = outs[2 * NSTREAM]
    ts = jnp.concatenate(
        [ts_parts[j][:, j * CHUNK:(j + 1) * CHUNK] for j in range(NSTREAM)], axis=1
    )
    se = jnp.concatenate(
        [se_parts[j][:, j * CHUNK:(j + 1) * CHUNK] for j in range(NSTREAM)], axis=1
    )
    return ts.T, se.T, counts.reshape(NUM_EXPERTS)


# speedup vs baseline: 1.0765x; 1.0587x over previous
import jax
import jax.numpy as jnp
from jax import lax
from jax.experimental import pallas as pl

DIM = 2048
NUM_EXPERTS = 16
TOP_K = 2
T = 16384
NSTREAM = 2
TT = 1024
CHUNK = T // NSTREAM


def _top2(scores, biased):
    iota = lax.broadcasted_iota(jnp.int32, (NUM_EXPERTS, TT), 0)
    neg_inf = jnp.float32(-jnp.inf)
    m1 = jnp.max(biased, axis=0, keepdims=True)
    idx1 = jnp.min(jnp.where(biased == m1, iota, NUM_EXPERTS), axis=0, keepdims=True)
    sel1 = iota == idx1
    s1 = jnp.max(jnp.where(sel1, scores, neg_inf), axis=0, keepdims=True)
    biased2 = jnp.where(sel1, neg_inf, biased)
    m2 = jnp.max(biased2, axis=0, keepdims=True)
    idx2 = jnp.min(jnp.where(biased2 == m2, iota, NUM_EXPERTS), axis=0, keepdims=True)
    sel2 = iota == idx2
    s2 = jnp.max(jnp.where(sel2, scores, neg_inf), axis=0, keepdims=True)
    ts = jnp.concatenate([s1, s2], axis=0)
    se = jnp.concatenate([idx1, idx2], axis=0)
    onehot = sel1.astype(jnp.float32) + sel2.astype(jnp.float32)
    return ts, se, onehot


def _router_body(*refs):
    x_refs = refs[:NSTREAM]
    w_ref, b_ref = refs[NSTREAM], refs[NSTREAM + 1]
    ts_refs = refs[NSTREAM + 2:2 * NSTREAM + 2]
    se_refs = refs[2 * NSTREAM + 2:3 * NSTREAM + 2]
    cnt_ref = refs[3 * NSTREAM + 2]
    i = pl.program_id(0)

    w = w_ref[...]
    b = b_ref[...]
    ones = jnp.ones((TT, 1), dtype=jnp.float32)
    cnt = jnp.zeros((NUM_EXPERTS, 1), dtype=jnp.float32)
    for j in range(NSTREAM):
        logits = lax.dot_general(
            w, x_refs[j][...],
            dimension_numbers=(((1,), (1,)), ((), ())),
            preferred_element_type=jnp.float32,
        )
        scores = jax.nn.sigmoid(logits)
        ts, se, onehot = _top2(scores, scores + b)
        ts_refs[j][...] = ts
        se_refs[j][...] = se
        cnt = cnt + lax.dot_general(
            onehot, ones,
            dimension_numbers=(((1,), (0,)), ((), ())),
            preferred_element_type=jnp.float32,
        )

    @pl.when(i == 0)
    def _init():
        cnt_ref[...] = cnt

    @pl.when(i > 0)
    def _acc():
        cnt_ref[...] += cnt


def kernel(x, W, expert_bias):
    bias2d = expert_bias.reshape(NUM_EXPERTS, 1)
    nb = CHUNK // TT
    grid = (nb,)

    def x_spec(j):
        return pl.BlockSpec((TT, DIM), lambda i, j=j: (j * nb + i, 0))

    def o_spec(j):
        return pl.BlockSpec((TOP_K, TT), lambda i, j=j: (0, j * nb + i))

    outs = pl.pallas_call(
        _router_body,
        grid=grid,
        in_specs=[x_spec(j) for j in range(NSTREAM)]
        + [
            pl.BlockSpec((NUM_EXPERTS, DIM), lambda i: (0, 0)),
            pl.BlockSpec((NUM_EXPERTS, 1), lambda i: (0, 0)),
        ],
        out_specs=[o_spec(j) for j in range(NSTREAM)]
        + [o_spec(j) for j in range(NSTREAM)]
        + [pl.BlockSpec((NUM_EXPERTS, 1), lambda i: (0, 0))],
        out_shape=[jax.ShapeDtypeStruct((TOP_K, T), jnp.float32)] * NSTREAM
        + [jax.ShapeDtypeStruct((TOP_K, T), jnp.int32)] * NSTREAM
        + [jax.ShapeDtypeStruct((NUM_EXPERTS, 1), jnp.float32)],
    )(*([x] * NSTREAM), W, bias2d)

    ts_parts = outs[:NSTREAM]
    se_parts = outs[NSTREAM:2 * NSTREAM]
    counts = outs[2 * NSTREAM]
    ts = jnp.concatenate(
        [ts_parts[j][:, j * CHUNK:(j + 1) * CHUNK] for j in range(NSTREAM)], axis=1
    )
    se = jnp.concatenate(
        [se_parts[j][:, j * CHUNK:(j + 1) * CHUNK] for j in range(NSTREAM)], axis=1
    )
    return ts.T, se.T, counts.reshape(NUM_EXPERTS)
